# baseline (speedup 1.0000x reference)
import functools

import jax
import jax.numpy as jnp
from jax import lax
from jax.experimental import pallas as pl
from jax.experimental.pallas import tpu as pltpu
from jax.experimental.pallas import tpu_sc as plsc

_NC = 2
_NS = 16
_NW = _NC * _NS
_L = 16
_CH = 128
_ZR = 8


def _pad_rows(N):
    return ((N + 2047) // 2048) * 2048


def _epw(E):
    return ((E + _NW * _CH - 1) // (_NW * _CH)) * _CH


_FRAC0 = 1.0


@functools.lru_cache(maxsize=None)
def _sc_segment_sum(N, D, E, with_cnt=False):
    EPW = _epw(E)
    Ep = EPW * _NW
    GR = _NS * _CH
    E0 = int(round(Ep * _FRAC0 / GR)) * GR
    EPW0 = E0 // _NS
    EPW1 = (Ep - E0) // _NS
    NCH0 = EPW0 // _CH
    NCH1 = EPW1 // _CH
    NP = _pad_rows(N)
    RPT = NP // _NS
    assert RPT % _ZR == 0 and NCH0 >= 3 and (NCH1 >= 3 or NCH1 == 0)

    mesh = plsc.VectorSubcoreMesh(core_axis_name="c", subcore_axis_name="s")

    def body(table, ei, sum_out, *rest):
        if with_cnt:
            (cnt_out, exv, rows_v, zrow_v, acc_sh, isem, gsem, ssem) = rest
        else:
            (exv, rows_v, zrow_v, acc_sh, isem, gsem, ssem) = rest
        cid = lax.axis_index("c")
        sid = lax.axis_index("s")
        wid = sid * _NC + cid
        row0 = sid * RPT

        for r in range(_ZR):
            for c in range(D // _L):
                zrow_v[r, pl.ds(c * _L, _L)] = jnp.zeros((_L,), jnp.float32)

        @pl.loop(0, RPT // _ZR)
        def zloop(i):
            pltpu.sync_copy(zrow_v, acc_sh.at[pl.ds(row0 + i * _ZR, _ZR)])

        if with_cnt:
            one = jnp.ones((_L,), jnp.float32)

            @pl.loop(0, _CH)
            def oinit(r):
                for c in range(D // _L):
                    rows_v[1, r, pl.ds(c * _L, _L)] = one

            plsc.subcore_barrier()

            cbase = wid * EPW
            NCHC = EPW // _CH

            def idx_src_c(j):
                return ei.at[:, pl.ds(cbase + j * _CH, _CH)]

            def cscatter(j):
                return (rows_v.at[1], acc_sh.at[exv.at[lax.rem(j, 4), 1]])

            pltpu.async_copy(idx_src_c(0), exv.at[0], isem)
            pltpu.async_copy(idx_src_c(1), exv.at[1], isem)

            @pl.loop(0, NCHC)
            def cloop(j):
                @pl.when(j >= 2)
                def drain_cscatter():
                    d_src, d_dst = cscatter(j - 2)
                    pltpu.make_async_copy(d_src, d_dst, ssem).wait()

                @pl.when(j + 2 < NCHC)
                def prefetch_cidx():
                    pltpu.async_copy(idx_src_c(j + 2),
                                     exv.at[lax.rem(j + 2, 4)], isem)

                pltpu.make_async_copy(idx_src_c(j), exv.at[lax.rem(j, 4)],
                                      isem).wait()
                c_src, c_dst = cscatter(j)
                pltpu.async_copy(c_src, c_dst, ssem, add=True)

            for dj in (NCHC - 2, NCHC - 1):
                d_src, d_dst = cscatter(dj)
                pltpu.make_async_copy(d_src, d_dst, ssem).wait()

            plsc.subcore_barrier()

            pltpu.sync_copy(acc_sh.at[pl.ds(row0, RPT)],
                            cnt_out.at[cid, pl.ds(row0, RPT)])

            @pl.loop(0, RPT // _ZR)
            def rezero(i):
                pltpu.sync_copy(zrow_v, acc_sh.at[pl.ds(row0 + i * _ZR, _ZR)])

        plsc.subcore_barrier()

        def run_edges(ebase, NCH):
            def idx_src(j):
                return ei.at[:, pl.ds(ebase + j * _CH, _CH)]

            H = _CH // 2

            def gather_h(j, slot, h):
                return (table.at[exv.at[slot, 0, pl.ds(h * H, H)]],
                        rows_v.at[lax.rem(j, 2), pl.ds(h * H, H)])

            def issue_gather(j, slot):
                for h in range(2):
                    g_src, g_dst = gather_h(j, slot, h)
                    pltpu.async_copy(g_src, g_dst, gsem)

            def wait_gather(j, slot):
                for h in range(2):
                    g_src, g_dst = gather_h(j, slot, h)
                    pltpu.make_async_copy(g_src, g_dst, gsem).wait()

            def scatter(j, slot):
                return (rows_v.at[lax.rem(j, 2)],
                        acc_sh.at[exv.at[slot, 1]])

            pltpu.async_copy(idx_src(0), exv.at[0], isem)
            pltpu.async_copy(idx_src(1), exv.at[1], isem)
            pltpu.make_async_copy(idx_src(0), exv.at[0], isem).wait()
            issue_gather(0, 0)

            @pl.loop(0, NCH)
            def eloop(j):
                s0 = lax.rem(j, 4)
                s1 = lax.rem(j + 1, 4)
                s2 = lax.rem(j + 2, 4)

                @pl.when(j >= 1)
                def drain_scatter():
                    d_src, d_dst = scatter(j - 1, lax.rem(j + 3, 4))
                    pltpu.make_async_copy(d_src, d_dst, ssem).wait()

                @pl.when(j + 2 < NCH)
                def prefetch_idx():
                    pltpu.async_copy(idx_src(j + 2), exv.at[s2], isem)

                @pl.when(j + 1 < NCH)
                def next_gather():
                    pltpu.make_async_copy(idx_src(j + 1), exv.at[s1],
                                          isem).wait()
                    issue_gather(j + 1, s1)

                wait_gather(j, s0)
                sc_src, sc_dst = scatter(j, s0)
                pltpu.async_copy(sc_src, sc_dst, ssem, add=True)

            d_src, d_dst = scatter(NCH - 1, (NCH - 1) % 4)
            pltpu.make_async_copy(d_src, d_dst, ssem).wait()

        @pl.when(cid == 0)
        def core0_edges():
            run_edges(sid * EPW0, NCH0)

        if NCH1 > 0:
            @pl.when(cid == 1)
            def core1_edges():
                run_edges(E0 + sid * EPW1, NCH1)

        plsc.subcore_barrier()

        pltpu.sync_copy(acc_sh.at[pl.ds(row0, RPT)],
                        sum_out.at[cid, pl.ds(row0, RPT)])

    outs = [jax.ShapeDtypeStruct((_NC, NP, D), jnp.float32)]
    if with_cnt:
        outs.append(jax.ShapeDtypeStruct((_NC, NP, D), jnp.float32))
    return pl.kernel(
        body,
        mesh=mesh,
        out_type=outs,
        scratch_types=[
            pltpu.VMEM((4, 2, _CH), jnp.int32),
            pltpu.VMEM((2, _CH, D), jnp.float32),
            pltpu.VMEM((_ZR, D), jnp.float32),
            pltpu.VMEM_SHARED((NP, D), jnp.float32),
            pltpu.SemaphoreType.DMA,
            pltpu.SemaphoreType.DMA,
            pltpu.SemaphoreType.DMA,
        ],
    )


@functools.lru_cache(maxsize=None)
def _tc_conv(N, D, BLK=512):
    def body(x_ref, s_ref, c_ref, rw_ref, rb_ref, relw_ref, o_ref):
        s = s_ref[0] + s_ref[1]
        cnt = c_ref[0, :, 0:1] + c_ref[1, :, 0:1]
        mean = s / jnp.maximum(cnt, 1.0)
        h = jnp.dot(x_ref[...], rw_ref[...], preferred_element_type=jnp.float32)
        h = h + rb_ref[...]
        h = h + jnp.dot(mean, relw_ref[...], preferred_element_type=jnp.float32)
        o_ref[...] = jnp.maximum(h, 0.0)

    return pl.pallas_call(
        body,
        grid=(pl.cdiv(N, BLK),),
        in_specs=[
            pl.BlockSpec((BLK, D), lambda i: (i, 0)),
            pl.BlockSpec((2, BLK, D), lambda i: (0, i, 0)),
            pl.BlockSpec((2, BLK, D), lambda i: (0, i, 0)),
            pl.BlockSpec((D, D), lambda i: (0, 0)),
            pl.BlockSpec((1, D), lambda i: (0, 0)),
            pl.BlockSpec((D, D), lambda i: (0, 0)),
        ],
        out_specs=pl.BlockSpec((BLK, D), lambda i: (i, 0)),
        out_shape=jax.ShapeDtypeStruct((N, D), jnp.float32),
    )


@functools.lru_cache(maxsize=None)
def _tc_final(N, D, C, BLK=512):
    def body(h1_ref, s_ref, c_ref, rw_ref, rb_ref, relw_ref, aw_ref,
             ow_ref, ob_ref, out_ref, alpha_ref):
        h1 = h1_ref[...]
        s = s_ref[0] + s_ref[1]
        cnt = c_ref[0, :, 0:1] + c_ref[1, :, 0:1]
        mean = s / jnp.maximum(cnt, 1.0)
        h2 = jnp.dot(h1, rw_ref[...], preferred_element_type=jnp.float32)
        h2 = h2 + rb_ref[...]
        h2 = h2 + jnp.dot(mean, relw_ref[...], preferred_element_type=jnp.float32)
        h2 = jnp.maximum(h2, 0.0)
        aw = aw_ref[...]
        s1 = jnp.sum(h1 * aw, axis=1, keepdims=True)
        s2 = jnp.sum(h2 * aw, axis=1, keepdims=True)
        m = jnp.maximum(s1, s2)
        e1 = jnp.exp(s1 - m)
        e2 = jnp.exp(s2 - m)
        z = e1 + e2
        a1 = e1 / z
        a2 = e2 / z
        h = a1 * h1 + a2 * h2
        out_ref[...] = (jnp.dot(h, ow_ref[...], preferred_element_type=jnp.float32)
                        + ob_ref[...])
        alpha_ref[...] = jnp.concatenate([a1, a2], axis=1)

    return pl.pallas_call(
        body,
        grid=(pl.cdiv(N, BLK),),
        in_specs=[
            pl.BlockSpec((BLK, D), lambda i: (i, 0)),
            pl.BlockSpec((2, BLK, D), lambda i: (0, i, 0)),
            pl.BlockSpec((2, BLK, D), lambda i: (0, i, 0)),
            pl.BlockSpec((D, D), lambda i: (0, 0)),
            pl.BlockSpec((1, D), lambda i: (0, 0)),
            pl.BlockSpec((D, D), lambda i: (0, 0)),
            pl.BlockSpec((1, D), lambda i: (0, 0)),
            pl.BlockSpec((D, C), lambda i: (0, 0)),
            pl.BlockSpec((1, C), lambda i: (0, 0)),
        ],
        out_specs=[
            pl.BlockSpec((BLK, C), lambda i: (i, 0)),
            pl.BlockSpec((BLK, 2), lambda i: (i, 0)),
        ],
        out_shape=[
            jax.ShapeDtypeStruct((N, C), jnp.float32),
            jax.ShapeDtypeStruct((N, 2), jnp.float32),
        ],
    )


def kernel(x, edge_index, rel_W1, root_W1, root_b1, rel_W2, root_W2, root_b2,
           att_w, out_W, out_b):
    N, D = x.shape
    E = edge_index.shape[1]
    C = out_W.shape[1]
    NP = _pad_rows(N)
    Ep = _epw(E) * _NW

    src = edge_index[0].astype(jnp.int32)
    dst = edge_index[1].astype(jnp.int32)
    pad = Ep - E
    src_p = jnp.concatenate([src, jnp.zeros((pad,), jnp.int32)])
    dst_p = jnp.concatenate([dst, jnp.full((pad,), NP - 1, jnp.int32)])
    ei = jnp.stack([src_p, dst_p])

    sum1, cnt = _sc_segment_sum(N, D, E, True)(x, ei)
    h1 = _tc_conv(N, D)(x, sum1, cnt, root_W1, root_b1.reshape(1, D), rel_W1)
    sum2 = _sc_segment_sum(N, D, E, False)(h1, ei)
    if isinstance(sum2, (list, tuple)):
        sum2 = sum2[0]
    out, alpha = _tc_final(N, D, C)(
        h1, sum2, cnt, root_W2, root_b2.reshape(1, D), rel_W2,
        att_w.reshape(1, D), out_W, out_b.reshape(1, C))
    return out, alpha


# speedup vs baseline: 1.3155x; 1.3155x over previous
import functools

import jax
import jax.numpy as jnp
from jax import lax
from jax.experimental import pallas as pl
from jax.experimental.pallas import tpu as pltpu
from jax.experimental.pallas import tpu_sc as plsc

_NC = 2
_NS = 16
_NW = _NC * _NS
_L = 16
_CH = 128
_ZR = 8


def _pad_rows(N):
    return ((N + 2047) // 2048) * 2048


def _epw(E):
    return ((E + _NW * _CH - 1) // (_NW * _CH)) * _CH


_FRAC0 = 0.78


@functools.lru_cache(maxsize=None)
def _sc_segment_sum(N, D, E, with_cnt=False):
    EPW = _epw(E)
    Ep = EPW * _NW
    GR = _NS * _CH
    E0 = int(round(Ep * _FRAC0 / GR)) * GR
    EPW0 = E0 // _NS
    EPW1 = (Ep - E0) // _NS
    NCH0 = EPW0 // _CH
    NCH1 = EPW1 // _CH
    NP = _pad_rows(N)
    RPT = NP // _NS
    assert RPT % _ZR == 0 and NCH0 >= 3 and (NCH1 >= 3 or NCH1 == 0)

    mesh = plsc.VectorSubcoreMesh(core_axis_name="c", subcore_axis_name="s")

    def body(table, ei, sum_out, *rest):
        if with_cnt:
            (cnt_out, exv, rows_v, zrow_v, acc_sh, isem, gsem, ssem) = rest
        else:
            (exv, rows_v, zrow_v, acc_sh, isem, gsem, ssem) = rest
        cid = lax.axis_index("c")
        sid = lax.axis_index("s")
        wid = sid * _NC + cid
        row0 = sid * RPT

        for r in range(_ZR):
            for c in range(D // _L):
                zrow_v[r, pl.ds(c * _L, _L)] = jnp.zeros((_L,), jnp.float32)

        @pl.loop(0, RPT // _ZR)
        def zloop(i):
            pltpu.sync_copy(zrow_v, acc_sh.at[pl.ds(row0 + i * _ZR, _ZR)])

        if with_cnt:
            one = jnp.ones((_L,), jnp.float32)

            @pl.loop(0, _CH)
            def oinit(r):
                for c in range(D // _L):
                    rows_v[1, r, pl.ds(c * _L, _L)] = one

            plsc.subcore_barrier()

            cbase = wid * EPW
            NCHC = EPW // _CH

            def idx_src_c(j):
                return ei.at[:, pl.ds(cbase + j * _CH, _CH)]

            def cscatter(j):
                return (rows_v.at[1], acc_sh.at[exv.at[lax.rem(j, 4), 1]])

            pltpu.async_copy(idx_src_c(0), exv.at[0], isem)
            pltpu.async_copy(idx_src_c(1), exv.at[1], isem)

            @pl.loop(0, NCHC)
            def cloop(j):
                @pl.when(j >= 2)
                def drain_cscatter():
                    d_src, d_dst = cscatter(j - 2)
                    pltpu.make_async_copy(d_src, d_dst, ssem).wait()

                @pl.when(j + 2 < NCHC)
                def prefetch_cidx():
                    pltpu.async_copy(idx_src_c(j + 2),
                                     exv.at[lax.rem(j + 2, 4)], isem)

                pltpu.make_async_copy(idx_src_c(j), exv.at[lax.rem(j, 4)],
                                      isem).wait()
                c_src, c_dst = cscatter(j)
                pltpu.async_copy(c_src, c_dst, ssem, add=True)

            for dj in (NCHC - 2, NCHC - 1):
                d_src, d_dst = cscatter(dj)
                pltpu.make_async_copy(d_src, d_dst, ssem).wait()

            plsc.subcore_barrier()

            pltpu.sync_copy(acc_sh.at[pl.ds(row0, RPT)],
                            cnt_out.at[cid, pl.ds(row0, RPT)])

            @pl.loop(0, RPT // _ZR)
            def rezero(i):
                pltpu.sync_copy(zrow_v, acc_sh.at[pl.ds(row0 + i * _ZR, _ZR)])

        plsc.subcore_barrier()

        def run_edges(ebase, NCH):
            def idx_src(j):
                return ei.at[:, pl.ds(ebase + j * _CH, _CH)]

            H = _CH // 2

            def gather_h(j, slot, h):
                return (table.at[exv.at[slot, 0, pl.ds(h * H, H)]],
                        rows_v.at[lax.rem(j, 2), pl.ds(h * H, H)])

            def issue_gather(j, slot):
                for h in range(2):
                    g_src, g_dst = gather_h(j, slot, h)
                    pltpu.async_copy(g_src, g_dst, gsem)

            def wait_gather(j, slot):
                for h in range(2):
                    g_src, g_dst = gather_h(j, slot, h)
                    pltpu.make_async_copy(g_src, g_dst, gsem).wait()

            def scatter(j, slot):
                return (rows_v.at[lax.rem(j, 2)],
                        acc_sh.at[exv.at[slot, 1]])

            pltpu.async_copy(idx_src(0), exv.at[0], isem)
            pltpu.async_copy(idx_src(1), exv.at[1], isem)
            pltpu.make_async_copy(idx_src(0), exv.at[0], isem).wait()
            issue_gather(0, 0)

            @pl.loop(0, NCH)
            def eloop(j):
                s0 = lax.rem(j, 4)
                s1 = lax.rem(j + 1, 4)
                s2 = lax.rem(j + 2, 4)

                @pl.when(j >= 1)
                def drain_scatter():
                    d_src, d_dst = scatter(j - 1, lax.rem(j + 3, 4))
                    pltpu.make_async_copy(d_src, d_dst, ssem).wait()

                @pl.when(j + 2 < NCH)
                def prefetch_idx():
                    pltpu.async_copy(idx_src(j + 2), exv.at[s2], isem)

                @pl.when(j + 1 < NCH)
                def next_gather():
                    pltpu.make_async_copy(idx_src(j + 1), exv.at[s1],
                                          isem).wait()
                    issue_gather(j + 1, s1)

                wait_gather(j, s0)
                sc_src, sc_dst = scatter(j, s0)
                pltpu.async_copy(sc_src, sc_dst, ssem, add=True)

            d_src, d_dst = scatter(NCH - 1, (NCH - 1) % 4)
            pltpu.make_async_copy(d_src, d_dst, ssem).wait()

        @pl.when(cid == 0)
        def core0_edges():
            run_edges(sid * EPW0, NCH0)

        if NCH1 > 0:
            @pl.when(cid == 1)
            def core1_edges():
                run_edges(E0 + sid * EPW1, NCH1)

        plsc.subcore_barrier()

        pltpu.sync_copy(acc_sh.at[pl.ds(row0, RPT)],
                        sum_out.at[cid, pl.ds(row0, RPT)])

    outs = [jax.ShapeDtypeStruct((_NC, NP, D), jnp.float32)]
    if with_cnt:
        outs.append(jax.ShapeDtypeStruct((_NC, NP, D), jnp.float32))
    return pl.kernel(
        body,
        mesh=mesh,
        out_type=outs,
        scratch_types=[
            pltpu.VMEM((4, 2, _CH), jnp.int32),
            pltpu.VMEM((2, _CH, D), jnp.float32),
            pltpu.VMEM((_ZR, D), jnp.float32),
            pltpu.VMEM_SHARED((NP, D), jnp.float32),
            pltpu.SemaphoreType.DMA,
            pltpu.SemaphoreType.DMA,
            pltpu.SemaphoreType.DMA,
        ],
    )


@functools.lru_cache(maxsize=None)
def _tc_conv(N, D, BLK=512):
    def body(x_ref, s_ref, c_ref, rw_ref, rb_ref, relw_ref, o_ref):
        s = s_ref[0] + s_ref[1]
        cnt = c_ref[0, :, 0:1] + c_ref[1, :, 0:1]
        mean = s / jnp.maximum(cnt, 1.0)
        h = jnp.dot(x_ref[...], rw_ref[...], preferred_element_type=jnp.float32)
        h = h + rb_ref[...]
        h = h + jnp.dot(mean, relw_ref[...], preferred_element_type=jnp.float32)
        o_ref[...] = jnp.maximum(h, 0.0)

    return pl.pallas_call(
        body,
        grid=(pl.cdiv(N, BLK),),
        in_specs=[
            pl.BlockSpec((BLK, D), lambda i: (i, 0)),
            pl.BlockSpec((2, BLK, D), lambda i: (0, i, 0)),
            pl.BlockSpec((2, BLK, D), lambda i: (0, i, 0)),
            pl.BlockSpec((D, D), lambda i: (0, 0)),
            pl.BlockSpec((1, D), lambda i: (0, 0)),
            pl.BlockSpec((D, D), lambda i: (0, 0)),
        ],
        out_specs=pl.BlockSpec((BLK, D), lambda i: (i, 0)),
        out_shape=jax.ShapeDtypeStruct((N, D), jnp.float32),
    )


@functools.lru_cache(maxsize=None)
def _tc_final(N, D, C, BLK=512):
    def body(h1_ref, s_ref, c_ref, rw_ref, rb_ref, relw_ref, aw_ref,
             ow_ref, ob_ref, out_ref, alpha_ref):
        h1 = h1_ref[...]
        s = s_ref[0] + s_ref[1]
        cnt = c_ref[0, :, 0:1] + c_ref[1, :, 0:1]
        mean = s / jnp.maximum(cnt, 1.0)
        h2 = jnp.dot(h1, rw_ref[...], preferred_element_type=jnp.float32)
        h2 = h2 + rb_ref[...]
        h2 = h2 + jnp.dot(mean, relw_ref[...], preferred_element_type=jnp.float32)
        h2 = jnp.maximum(h2, 0.0)
        aw = aw_ref[...]
        s1 = jnp.sum(h1 * aw, axis=1, keepdims=True)
        s2 = jnp.sum(h2 * aw, axis=1, keepdims=True)
        m = jnp.maximum(s1, s2)
        e1 = jnp.exp(s1 - m)
        e2 = jnp.exp(s2 - m)
        z = e1 + e2
        a1 = e1 / z
        a2 = e2 / z
        h = a1 * h1 + a2 * h2
        out_ref[...] = (jnp.dot(h, ow_ref[...], preferred_element_type=jnp.float32)
                        + ob_ref[...])
        alpha_ref[...] = jnp.concatenate([a1, a2], axis=1)

    return pl.pallas_call(
        body,
        grid=(pl.cdiv(N, BLK),),
        in_specs=[
            pl.BlockSpec((BLK, D), lambda i: (i, 0)),
            pl.BlockSpec((2, BLK, D), lambda i: (0, i, 0)),
            pl.BlockSpec((2, BLK, D), lambda i: (0, i, 0)),
            pl.BlockSpec((D, D), lambda i: (0, 0)),
            pl.BlockSpec((1, D), lambda i: (0, 0)),
            pl.BlockSpec((D, D), lambda i: (0, 0)),
            pl.BlockSpec((1, D), lambda i: (0, 0)),
            pl.BlockSpec((D, C), lambda i: (0, 0)),
            pl.BlockSpec((1, C), lambda i: (0, 0)),
        ],
        out_specs=[
            pl.BlockSpec((BLK, C), lambda i: (i, 0)),
            pl.BlockSpec((BLK, 2), lambda i: (i, 0)),
        ],
        out_shape=[
            jax.ShapeDtypeStruct((N, C), jnp.float32),
            jax.ShapeDtypeStruct((N, 2), jnp.float32),
        ],
    )


def kernel(x, edge_index, rel_W1, root_W1, root_b1, rel_W2, root_W2, root_b2,
           att_w, out_W, out_b):
    N, D = x.shape
    E = edge_index.shape[1]
    C = out_W.shape[1]
    NP = _pad_rows(N)
    Ep = _epw(E) * _NW

    src = edge_index[0].astype(jnp.int32)
    dst = edge_index[1].astype(jnp.int32)
    pad = Ep - E
    src_p = jnp.concatenate([src, jnp.zeros((pad,), jnp.int32)])
    dst_p = jnp.concatenate([dst, jnp.full((pad,), NP - 1, jnp.int32)])
    ei = jnp.stack([src_p, dst_p])

    sum1, cnt = _sc_segment_sum(N, D, E, True)(x, ei)
    h1 = _tc_conv(N, D)(x, sum1, cnt, root_W1, root_b1.reshape(1, D), rel_W1)
    sum2 = _sc_segment_sum(N, D, E, False)(h1, ei)
    if isinstance(sum2, (list, tuple)):
        sum2 = sum2[0]
    out, alpha = _tc_final(N, D, C)(
        h1, sum2, cnt, root_W2, root_b2.reshape(1, D), rel_W2,
        att_w.reshape(1, D), out_W, out_b.reshape(1, C))
    return out, alpha


# speedup vs baseline: 1.3472x; 1.0241x over previous
import functools

import jax
import jax.numpy as jnp
from jax import lax
from jax.experimental import pallas as pl
from jax.experimental.pallas import tpu as pltpu
from jax.experimental.pallas import tpu_sc as plsc

_NC = 2
_NS = 16
_NW = _NC * _NS
_L = 16
_CH = 128
_ZR = 8


def _pad_rows(N):
    return ((N + 2047) // 2048) * 2048


def _epw(E):
    return ((E + _NW * _CH - 1) // (_NW * _CH)) * _CH


_FRAC0 = 0.84


@functools.lru_cache(maxsize=None)
def _sc_segment_sum(N, D, E, with_cnt=False):
    EPW = _epw(E)
    Ep = EPW * _NW
    GR = _NS * _CH
    E0 = int(round(Ep * _FRAC0 / GR)) * GR
    EPW0 = E0 // _NS
    EPW1 = (Ep - E0) // _NS
    NCH0 = EPW0 // _CH
    NCH1 = EPW1 // _CH
    NP = _pad_rows(N)
    RPT = NP // _NS
    assert RPT % _ZR == 0 and NCH0 >= 3 and (NCH1 >= 3 or NCH1 == 0)

    mesh = plsc.VectorSubcoreMesh(core_axis_name="c", subcore_axis_name="s")

    def body(table, ei, sum_out, *rest):
        if with_cnt:
            (cnt_out, exv, rows_v, zrow_v, acc_sh, isem, gsem, ssem) = rest
        else:
            (exv, rows_v, zrow_v, acc_sh, isem, gsem, ssem) = rest
        cid = lax.axis_index("c")
        sid = lax.axis_index("s")
        wid = sid * _NC + cid
        row0 = sid * RPT

        for r in range(_ZR):
            for c in range(D // _L):
                zrow_v[r, pl.ds(c * _L, _L)] = jnp.zeros((_L,), jnp.float32)

        @pl.loop(0, RPT // _ZR)
        def zloop(i):
            pltpu.sync_copy(zrow_v, acc_sh.at[pl.ds(row0 + i * _ZR, _ZR)])

        if with_cnt:
            one = jnp.ones((_L,), jnp.float32)

            @pl.loop(0, _CH)
            def oinit(r):
                for c in range(D // _L):
                    rows_v[1, r, pl.ds(c * _L, _L)] = one

            plsc.subcore_barrier()

            cbase = wid * EPW
            NCHC = EPW // _CH

            def idx_src_c(j):
                return ei.at[:, pl.ds(cbase + j * _CH, _CH)]

            def cscatter(j):
                return (rows_v.at[1], acc_sh.at[exv.at[lax.rem(j, 4), 1]])

            pltpu.async_copy(idx_src_c(0), exv.at[0], isem)
            pltpu.async_copy(idx_src_c(1), exv.at[1], isem)

            @pl.loop(0, NCHC)
            def cloop(j):
                @pl.when(j >= 2)
                def drain_cscatter():
                    d_src, d_dst = cscatter(j - 2)
                    pltpu.make_async_copy(d_src, d_dst, ssem).wait()

                @pl.when(j + 2 < NCHC)
                def prefetch_cidx():
                    pltpu.async_copy(idx_src_c(j + 2),
                                     exv.at[lax.rem(j + 2, 4)], isem)

                pltpu.make_async_copy(idx_src_c(j), exv.at[lax.rem(j, 4)],
                                      isem).wait()
                c_src, c_dst = cscatter(j)
                pltpu.async_copy(c_src, c_dst, ssem, add=True)

            for dj in (NCHC - 2, NCHC - 1):
                d_src, d_dst = cscatter(dj)
                pltpu.make_async_copy(d_src, d_dst, ssem).wait()

            plsc.subcore_barrier()

            pltpu.sync_copy(acc_sh.at[pl.ds(row0, RPT)],
                            cnt_out.at[cid, pl.ds(row0, RPT)])

            @pl.loop(0, RPT // _ZR)
            def rezero(i):
                pltpu.sync_copy(zrow_v, acc_sh.at[pl.ds(row0 + i * _ZR, _ZR)])

        plsc.subcore_barrier()

        def run_edges(ebase, NCH):
            def idx_src(j):
                return ei.at[:, pl.ds(ebase + j * _CH, _CH)]

            H = _CH // 2

            def gather_h(j, slot, h):
                return (table.at[exv.at[slot, 0, pl.ds(h * H, H)]],
                        rows_v.at[lax.rem(j, 2), pl.ds(h * H, H)])

            def issue_gather(j, slot):
                for h in range(2):
                    g_src, g_dst = gather_h(j, slot, h)
                    pltpu.async_copy(g_src, g_dst, gsem)

            def wait_gather(j, slot):
                for h in range(2):
                    g_src, g_dst = gather_h(j, slot, h)
                    pltpu.make_async_copy(g_src, g_dst, gsem).wait()

            def scatter(j, slot):
                return (rows_v.at[lax.rem(j, 2)],
                        acc_sh.at[exv.at[slot, 1]])

            pltpu.async_copy(idx_src(0), exv.at[0], isem)
            pltpu.async_copy(idx_src(1), exv.at[1], isem)
            pltpu.make_async_copy(idx_src(0), exv.at[0], isem).wait()
            issue_gather(0, 0)

            @pl.loop(0, NCH)
            def eloop(j):
                s0 = lax.rem(j, 4)
                s1 = lax.rem(j + 1, 4)
                s2 = lax.rem(j + 2, 4)

                @pl.when(j >= 1)
                def drain_scatter():
                    d_src, d_dst = scatter(j - 1, lax.rem(j + 3, 4))
                    pltpu.make_async_copy(d_src, d_dst, ssem).wait()

                @pl.when(j + 2 < NCH)
                def prefetch_idx():
                    pltpu.async_copy(idx_src(j + 2), exv.at[s2], isem)

                @pl.when(j + 1 < NCH)
                def next_gather():
                    pltpu.make_async_copy(idx_src(j + 1), exv.at[s1],
                                          isem).wait()
                    issue_gather(j + 1, s1)

                wait_gather(j, s0)
                sc_src, sc_dst = scatter(j, s0)
                pltpu.async_copy(sc_src, sc_dst, ssem, add=True)

            d_src, d_dst = scatter(NCH - 1, (NCH - 1) % 4)
            pltpu.make_async_copy(d_src, d_dst, ssem).wait()

        @pl.when(cid == 0)
        def core0_edges():
            run_edges(sid * EPW0, NCH0)

        if NCH1 > 0:
            @pl.when(cid == 1)
            def core1_edges():
                run_edges(E0 + sid * EPW1, NCH1)

        plsc.subcore_barrier()

        pltpu.sync_copy(acc_sh.at[pl.ds(row0, RPT)],
                        sum_out.at[cid, pl.ds(row0, RPT)])

    outs = [jax.ShapeDtypeStruct((_NC, NP, D), jnp.float32)]
    if with_cnt:
        outs.append(jax.ShapeDtypeStruct((_NC, NP, D), jnp.float32))
    return pl.kernel(
        body,
        mesh=mesh,
        out_type=outs,
        scratch_types=[
            pltpu.VMEM((4, 2, _CH), jnp.int32),
            pltpu.VMEM((2, _CH, D), jnp.float32),
            pltpu.VMEM((_ZR, D), jnp.float32),
            pltpu.VMEM_SHARED((NP, D), jnp.float32),
            pltpu.SemaphoreType.DMA,
            pltpu.SemaphoreType.DMA,
            pltpu.SemaphoreType.DMA,
        ],
    )


@functools.lru_cache(maxsize=None)
def _tc_conv(N, D, BLK=512):
    def body(x_ref, s_ref, c_ref, rw_ref, rb_ref, relw_ref, o_ref):
        s = s_ref[0] + s_ref[1]
        cnt = c_ref[0, :, 0:1] + c_ref[1, :, 0:1]
        mean = s / jnp.maximum(cnt, 1.0)
        h = jnp.dot(x_ref[...], rw_ref[...], preferred_element_type=jnp.float32)
        h = h + rb_ref[...]
        h = h + jnp.dot(mean, relw_ref[...], preferred_element_type=jnp.float32)
        o_ref[...] = jnp.maximum(h, 0.0)

    return pl.pallas_call(
        body,
        grid=(pl.cdiv(N, BLK),),
        in_specs=[
            pl.BlockSpec((BLK, D), lambda i: (i, 0)),
            pl.BlockSpec((2, BLK, D), lambda i: (0, i, 0)),
            pl.BlockSpec((2, BLK, D), lambda i: (0, i, 0)),
            pl.BlockSpec((D, D), lambda i: (0, 0)),
            pl.BlockSpec((1, D), lambda i: (0, 0)),
            pl.BlockSpec((D, D), lambda i: (0, 0)),
        ],
        out_specs=pl.BlockSpec((BLK, D), lambda i: (i, 0)),
        out_shape=jax.ShapeDtypeStruct((N, D), jnp.float32),
    )


@functools.lru_cache(maxsize=None)
def _tc_final(N, D, C, BLK=512):
    def body(h1_ref, s_ref, c_ref, rw_ref, rb_ref, relw_ref, aw_ref,
             ow_ref, ob_ref, out_ref, alpha_ref):
        h1 = h1_ref[...]
        s = s_ref[0] + s_ref[1]
        cnt = c_ref[0, :, 0:1] + c_ref[1, :, 0:1]
        mean = s / jnp.maximum(cnt, 1.0)
        h2 = jnp.dot(h1, rw_ref[...], preferred_element_type=jnp.float32)
        h2 = h2 + rb_ref[...]
        h2 = h2 + jnp.dot(mean, relw_ref[...], preferred_element_type=jnp.float32)
        h2 = jnp.maximum(h2, 0.0)
        aw = aw_ref[...]
        s1 = jnp.sum(h1 * aw, axis=1, keepdims=True)
        s2 = jnp.sum(h2 * aw, axis=1, keepdims=True)
        m = jnp.maximum(s1, s2)
        e1 = jnp.exp(s1 - m)
        e2 = jnp.exp(s2 - m)
        z = e1 + e2
        a1 = e1 / z
        a2 = e2 / z
        h = a1 * h1 + a2 * h2
        out_ref[...] = (jnp.dot(h, ow_ref[...], preferred_element_type=jnp.float32)
                        + ob_ref[...])
        alpha_ref[...] = jnp.concatenate([a1, a2], axis=1)

    return pl.pallas_call(
        body,
        grid=(pl.cdiv(N, BLK),),
        in_specs=[
            pl.BlockSpec((BLK, D), lambda i: (i, 0)),
            pl.BlockSpec((2, BLK, D), lambda i: (0, i, 0)),
            pl.BlockSpec((2, BLK, D), lambda i: (0, i, 0)),
            pl.BlockSpec((D, D), lambda i: (0, 0)),
            pl.BlockSpec((1, D), lambda i: (0, 0)),
            pl.BlockSpec((D, D), lambda i: (0, 0)),
            pl.BlockSpec((1, D), lambda i: (0, 0)),
            pl.BlockSpec((D, C), lambda i: (0, 0)),
            pl.BlockSpec((1, C), lambda i: (0, 0)),
        ],
        out_specs=[
            pl.BlockSpec((BLK, C), lambda i: (i, 0)),
            pl.BlockSpec((BLK, 2), lambda i: (i, 0)),
        ],
        out_shape=[
            jax.ShapeDtypeStruct((N, C), jnp.float32),
            jax.ShapeDtypeStruct((N, 2), jnp.float32),
        ],
    )


def kernel(x, edge_index, rel_W1, root_W1, root_b1, rel_W2, root_W2, root_b2,
           att_w, out_W, out_b):
    N, D = x.shape
    E = edge_index.shape[1]
    C = out_W.shape[1]
    NP = _pad_rows(N)
    Ep = _epw(E) * _NW

    src = edge_index[0].astype(jnp.int32)
    dst = edge_index[1].astype(jnp.int32)
    pad = Ep - E
    src_p = jnp.concatenate([src, jnp.zeros((pad,), jnp.int32)])
    dst_p = jnp.concatenate([dst, jnp.full((pad,), NP - 1, jnp.int32)])
    ei = jnp.stack([src_p, dst_p])

    sum1, cnt = _sc_segment_sum(N, D, E, True)(x, ei)
    h1 = _tc_conv(N, D)(x, sum1, cnt, root_W1, root_b1.reshape(1, D), rel_W1)
    sum2 = _sc_segment_sum(N, D, E, False)(h1, ei)
    if isinstance(sum2, (list, tuple)):
        sum2 = sum2[0]
    out, alpha = _tc_final(N, D, C)(
        h1, sum2, cnt, root_W2, root_b2.reshape(1, D), rel_W2,
        att_w.reshape(1, D), out_W, out_b.reshape(1, C))
    return out, alpha
